# baseline (speedup 1.0000x reference)
import functools

import jax
import jax.numpy as jnp
from jax import lax
from jax.experimental import pallas as pl
from jax.experimental.pallas import tpu as pltpu
from jax.experimental.pallas import tpu_sc as plsc

NC = 2
NS = 16
NW = NC * NS
L = 16

BATCH = 4096
HIST = 200
CHUNK = 104
OFF2 = HIST - CHUNK
EMBED = 32
B_PER_W = BATCH // NW
DEPTH = 4
NBLK = HIST // L
TAIL_OFF = HIST - L


def _nbow_kernel(idxs_hbm, emb_hbm, tw_hbm, out_hbm,
                 idx_slab, w_b, rows_b, out_slab, sem_e, sem_w):
    wid = lax.axis_index("s") * NC + lax.axis_index("c")
    base = wid * B_PER_W

    pltpu.sync_copy(idxs_hbm.at[pl.ds(base * HIST, B_PER_W * HIST)],
                    idx_slab)

    def gathers(row, b):
        cps = []
        for off in (0, OFF2):
            cps.append(pltpu.make_async_copy(
                emb_hbm.at[idx_slab.at[pl.ds(row * HIST + off, CHUNK)]],
                rows_b.at[b, pl.ds(off, CHUNK)], sem_e.at[b]))
            cps.append(pltpu.make_async_copy(
                tw_hbm.at[idx_slab.at[pl.ds(row * HIST + off, CHUNK)]],
                w_b.at[b, pl.ds(off, CHUNK)], sem_w.at[b]))
        return cps

    def issue(row, b):
        for cp in gathers(row, b):
            cp.start()

    def wait(row, b):
        for cp in gathers(row, b):
            cp.wait()

    def compute(row, b):
        for off in [k * L for k in range(NBLK)] + [TAIL_OFF]:
            sl = pl.ds(off, L)
            iv = idx_slab[pl.ds(row * HIST + off, L)]
            w_b[b, sl] = jnp.where(iv != 0, w_b[b, sl], 0.0)

        def fma_block(wv, j0, jjs, a0, a1):
            for jj in jjs:
                j = j0 + jj
                ws = wv[jj]
                a0 = a0 + ws * rows_b[b, j, pl.ds(0, L)]
                a1 = a1 + ws * rows_b[b, j, pl.ds(L, L)]
            return a0, a1

        def fma_body(blk, carry):
            a0, a1 = carry
            wv = w_b[b, pl.ds(blk * L, L)]
            return fma_block(wv, blk * L, range(L), a0, a1)

        zero = jnp.zeros((L,), jnp.float32)
        a0, a1 = lax.fori_loop(0, NBLK, fma_body, (zero, zero))
        wv = w_b[b, pl.ds(TAIL_OFF, L)]
        a0, a1 = fma_block(wv, TAIL_OFF, range(L // 2, L), a0, a1)
        out_slab[row, pl.ds(0, L)] = a0
        out_slab[row, pl.ds(L, L)] = a1

    for b in range(DEPTH):
        issue(b, b)

    def outer(g, _):
        for b in range(DEPTH):
            row = g * DEPTH + b
            wait(row, b)
            compute(row, b)
            nxt = row + DEPTH

            @pl.when(nxt < B_PER_W)
            def _():
                issue(nxt, b)
        return 0

    lax.fori_loop(0, B_PER_W // DEPTH, outer, 0)
    pltpu.sync_copy(out_slab, out_hbm.at[pl.ds(base, B_PER_W)])


@jax.jit
def kernel(idxs, embedding, token_weights):
    idxs_flat = idxs.reshape(-1)

    mesh = plsc.VectorSubcoreMesh(core_axis_name="c", subcore_axis_name="s")
    k = functools.partial(
        pl.kernel,
        out_type=jax.ShapeDtypeStruct((BATCH, EMBED), jnp.float32),
        mesh=mesh,
        scratch_types=[
            pltpu.VMEM((B_PER_W * HIST,), jnp.int32),
            pltpu.VMEM((DEPTH, HIST), jnp.float32),
            pltpu.VMEM((DEPTH, HIST, EMBED), jnp.float32),
            pltpu.VMEM((B_PER_W, EMBED), jnp.float32),
            pltpu.SemaphoreType.DMA((DEPTH,)),
            pltpu.SemaphoreType.DMA((DEPTH,)),
        ],
        compiler_params=pltpu.CompilerParams(use_tc_tiling_on_sc=False),
    )(_nbow_kernel)
    return k(idxs_flat, embedding, token_weights)


# speedup vs baseline: 1.0011x; 1.0011x over previous
import functools

import jax
import jax.numpy as jnp
from jax import lax
from jax.experimental import pallas as pl
from jax.experimental.pallas import tpu as pltpu
from jax.experimental.pallas import tpu_sc as plsc

NC = 2
NS = 16
NW = NC * NS
L = 16

BATCH = 4096
HIST = 200
CHUNK = 104
OFF2 = HIST - CHUNK
EMBED = 32
B_PER_W = BATCH // NW
DEPTH = 4
NBLK = HIST // L
TAIL_OFF = HIST - L


def _nbow_kernel(idxs_hbm, emb_hbm, tw_hbm, out_hbm,
                 idx_slab, w_b, rows_b, out_slab, sem_e, sem_w):
    wid = lax.axis_index("s") * NC + lax.axis_index("c")
    base = wid * B_PER_W

    pltpu.sync_copy(idxs_hbm.at[pl.ds(base * HIST, B_PER_W * HIST)],
                    idx_slab)

    def gathers(row, b):
        cps = []
        for off in (0, OFF2):
            cps.append(pltpu.make_async_copy(
                emb_hbm.at[idx_slab.at[pl.ds(row * HIST + off, CHUNK)]],
                rows_b.at[b, pl.ds(off, CHUNK)], sem_e.at[b]))
            cps.append(pltpu.make_async_copy(
                tw_hbm.at[idx_slab.at[pl.ds(row * HIST + off, CHUNK)]],
                w_b.at[b, pl.ds(off, CHUNK)], sem_w.at[b]))
        return cps

    def issue(row, b):
        for cp in gathers(row, b):
            cp.start()

    def wait(row, b):
        for cp in gathers(row, b):
            cp.wait()

    def compute(row, b):
        for off in [k * L for k in range(NBLK)] + [TAIL_OFF]:
            sl = pl.ds(off, L)
            iv = idx_slab[pl.ds(row * HIST + off, L)]
            w_b[b, sl] = jnp.where(iv != 0, w_b[b, sl], 0.0)

        def fma_block(wv, j0, jjs, a0, a1):
            for jj in jjs:
                j = j0 + jj
                ws = wv[jj]
                a0 = a0 + ws * rows_b[b, j, pl.ds(0, L)]
                a1 = a1 + ws * rows_b[b, j, pl.ds(L, L)]
            return a0, a1

        def fma_body(blk, carry):
            a0, a1 = carry
            wv = w_b[b, pl.ds(blk * L, L)]
            return fma_block(wv, blk * L, range(L), a0, a1)

        zero = jnp.zeros((L,), jnp.float32)
        a0, a1 = lax.fori_loop(0, NBLK, fma_body, (zero, zero))
        wv = w_b[b, pl.ds(TAIL_OFF, L)]
        a0, a1 = fma_block(wv, TAIL_OFF, range(L // 2, L), a0, a1)
        out_slab[pl.ds(row * EMBED, L)] = a0
        out_slab[pl.ds(row * EMBED + L, L)] = a1

    for b in range(DEPTH):
        issue(b, b)

    def outer(g, _):
        for b in range(DEPTH):
            row = g * DEPTH + b
            wait(row, b)
            compute(row, b)
            nxt = row + DEPTH

            @pl.when(nxt < B_PER_W)
            def _():
                issue(nxt, b)
        return 0

    lax.fori_loop(0, B_PER_W // DEPTH, outer, 0)
    pltpu.sync_copy(out_slab, out_hbm.at[pl.ds(base * EMBED, B_PER_W * EMBED)])


@jax.jit
def kernel(idxs, embedding, token_weights):
    idxs_flat = idxs.reshape(-1)

    mesh = plsc.VectorSubcoreMesh(core_axis_name="c", subcore_axis_name="s")
    k = functools.partial(
        pl.kernel,
        out_type=jax.ShapeDtypeStruct((BATCH * EMBED,), jnp.float32),
        mesh=mesh,
        scratch_types=[
            pltpu.VMEM((B_PER_W * HIST,), jnp.int32),
            pltpu.VMEM((DEPTH, HIST), jnp.float32),
            pltpu.VMEM((DEPTH, HIST, EMBED), jnp.float32),
            pltpu.VMEM((B_PER_W * EMBED,), jnp.float32),
            pltpu.SemaphoreType.DMA((DEPTH,)),
            pltpu.SemaphoreType.DMA((DEPTH,)),
        ],
        compiler_params=pltpu.CompilerParams(use_tc_tiling_on_sc=False),
    )(_nbow_kernel)
    return k(idxs_flat, embedding, token_weights).reshape(BATCH, EMBED)


# speedup vs baseline: 1.0036x; 1.0025x over previous
import functools

import jax
import jax.numpy as jnp
from jax import lax
from jax.experimental import pallas as pl
from jax.experimental.pallas import tpu as pltpu
from jax.experimental.pallas import tpu_sc as plsc

NC = 2
NS = 16
NW = NC * NS
L = 16

BATCH = 4096
HIST = 200
LANEW = 128
BW = HIST - LANEW
EMBED = 32
B_PER_W = BATCH // NW
DEPTH = 4
NBLK_A = LANEW // L
NBLK_B = BW // L
TAIL_B = BW - L


def _nbow_kernel(ia_hbm, ib_hbm, emb_hbm, tw_hbm, out_hbm,
                 slab_a, slab_b, w_b, rows_b, out_slab, sem_e, sem_w):
    wid = lax.axis_index("s") * NC + lax.axis_index("c")
    base = wid * B_PER_W

    pltpu.sync_copy(ia_hbm.at[pl.ds(base, B_PER_W)], slab_a)
    pltpu.sync_copy(ib_hbm.at[pl.ds(base, B_PER_W)], slab_b)

    def gathers(row, b):
        ia = slab_a.at[row, pl.ds(0, LANEW)]
        ib = slab_b.at[row, pl.ds(0, BW)]
        return [
            pltpu.make_async_copy(
                emb_hbm.at[ia], rows_b.at[b, pl.ds(0, LANEW)], sem_e.at[b]),
            pltpu.make_async_copy(
                emb_hbm.at[ib], rows_b.at[b, pl.ds(LANEW, BW)], sem_e.at[b]),
            pltpu.make_async_copy(
                tw_hbm.at[ia], w_b.at[b, pl.ds(0, LANEW)], sem_w.at[b]),
            pltpu.make_async_copy(
                tw_hbm.at[ib], w_b.at[b, pl.ds(LANEW, BW)], sem_w.at[b]),
        ]

    def issue(row, b):
        for cp in gathers(row, b):
            cp.start()

    def wait(row, b):
        for cp in gathers(row, b):
            cp.wait()

    def compute(row, b):
        blocks = [(slab_a, k * L, k * L) for k in range(NBLK_A)]
        blocks += [(slab_b, k * L, LANEW + k * L) for k in range(NBLK_B)]
        blocks += [(slab_b, TAIL_B, LANEW + TAIL_B)]
        for slab, ioff, woff in blocks:
            iv = slab[row, pl.ds(ioff, L)]
            sl = pl.ds(woff, L)
            w_b[b, sl] = jnp.where(iv != 0, w_b[b, sl], 0.0)

        def fma_block(wv, j0, jjs, a0, a1):
            for jj in jjs:
                j = j0 + jj
                ws = wv[jj]
                a0 = a0 + ws * rows_b[b, j, pl.ds(0, L)]
                a1 = a1 + ws * rows_b[b, j, pl.ds(L, L)]
            return a0, a1

        def fma_body(blk, carry):
            a0, a1 = carry
            wv = w_b[b, pl.ds(blk * L, L)]
            return fma_block(wv, blk * L, range(L), a0, a1)

        zero = jnp.zeros((L,), jnp.float32)
        a0, a1 = lax.fori_loop(0, NBLK_A + NBLK_B, fma_body, (zero, zero))
        wv = w_b[b, pl.ds(LANEW + TAIL_B, L)]
        a0, a1 = fma_block(wv, LANEW + TAIL_B, range(L // 2, L), a0, a1)
        out_slab[pl.ds(row * EMBED, L)] = a0
        out_slab[pl.ds(row * EMBED + L, L)] = a1

    for b in range(DEPTH):
        issue(b, b)

    def outer(g, _):
        for b in range(DEPTH):
            row = g * DEPTH + b
            wait(row, b)
            compute(row, b)
            nxt = row + DEPTH

            @pl.when(nxt < B_PER_W)
            def _():
                issue(nxt, b)
        return 0

    lax.fori_loop(0, B_PER_W // DEPTH, outer, 0)
    pltpu.sync_copy(out_slab, out_hbm.at[pl.ds(base * EMBED, B_PER_W * EMBED)])


@jax.jit
def kernel(idxs, embedding, token_weights):
    ia2 = lax.optimization_barrier(jnp.maximum(idxs[:, :LANEW], 0))
    ib2 = lax.optimization_barrier(jnp.maximum(jnp.concatenate(
        [idxs[:, LANEW:], jnp.zeros((BATCH, LANEW - BW), jnp.int32)],
        axis=1), 0))

    mesh = plsc.VectorSubcoreMesh(core_axis_name="c", subcore_axis_name="s")
    k = functools.partial(
        pl.kernel,
        out_type=jax.ShapeDtypeStruct((BATCH * EMBED,), jnp.float32),
        mesh=mesh,
        scratch_types=[
            pltpu.VMEM((B_PER_W, LANEW), jnp.int32),
            pltpu.VMEM((B_PER_W, LANEW), jnp.int32),
            pltpu.VMEM((DEPTH, HIST), jnp.float32),
            pltpu.VMEM((DEPTH, HIST, EMBED), jnp.float32),
            pltpu.VMEM((B_PER_W * EMBED,), jnp.float32),
            pltpu.SemaphoreType.DMA((DEPTH,)),
            pltpu.SemaphoreType.DMA((DEPTH,)),
        ],
        compiler_params=pltpu.CompilerParams(use_tc_tiling_on_sc=False),
    )(_nbow_kernel)
    return k(ia2, ib2, embedding, token_weights).reshape(BATCH, EMBED)
